# Initial kernel scaffold; baseline (speedup 1.0000x reference)
import jax
import jax.numpy as jnp
from jax.experimental import pallas as pl


def kernel(x):
    raise NotImplementedError("write your pallas kernel here")


# speedup vs baseline: 2.2023x; 2.2023x over previous
import jax
import jax.numpy as jnp
from jax.experimental import pallas as pl


def _cumsum_body(x_ref, o_ref):
    acc = x_ref[...]
    n = acc.shape[1]
    k = 1
    while k < n:
        z = jnp.zeros((acc.shape[0], k), acc.dtype)
        acc = acc + jnp.concatenate([z, acc[:, :-k]], axis=1)
        k *= 2
    o_ref[...] = acc


def kernel(x):
    m, n = x.shape
    r = 256
    return pl.pallas_call(
        _cumsum_body,
        grid=(m // r,),
        in_specs=[pl.BlockSpec((r, n), lambda i: (i, 0))],
        out_specs=pl.BlockSpec((r, n), lambda i: (i, 0)),
        out_shape=jax.ShapeDtypeStruct((m, n), x.dtype),
    )(x)


# speedup vs baseline: 5.0736x; 2.3038x over previous
import functools

import jax
import jax.numpy as jnp
from jax.experimental import pallas as pl

_CHUNK = 128


def _cumsum_body(x_ref, t_ref, d_ref, e_ref, o_ref):
    x = x_ref[...]
    t = t_ref[...]
    d = d_ref[...]
    e = e_ref[...]
    n = x.shape[1]
    nchunks = n // _CHUNK
    dot = functools.partial(
        jax.lax.dot, preferred_element_type=jnp.float32)
    ex = dot(dot(x, d), e)
    parts = [
        dot(x[:, i * _CHUNK:(i + 1) * _CHUNK], t) for i in range(nchunks)
    ]
    y = jnp.concatenate(parts, axis=1)
    o_ref[...] = y + ex


def kernel(x):
    m, n = x.shape
    r = 256
    nchunks = n // _CHUNK

    kk = jnp.arange(_CHUNK)
    t = (kk[:, None] <= kk[None, :]).astype(jnp.float32)
    krange = jnp.arange(n)
    crange = jnp.arange(nchunks)
    d = ((krange[:, None] // _CHUNK) < crange[None, :]).astype(jnp.float32)
    e = (crange[:, None] == (krange[None, :] // _CHUNK)).astype(jnp.float32)

    return pl.pallas_call(
        _cumsum_body,
        grid=(m // r,),
        in_specs=[
            pl.BlockSpec((r, n), lambda i: (i, 0)),
            pl.BlockSpec((_CHUNK, _CHUNK), lambda i: (0, 0)),
            pl.BlockSpec((n, nchunks), lambda i: (0, 0)),
            pl.BlockSpec((nchunks, n), lambda i: (0, 0)),
        ],
        out_specs=pl.BlockSpec((r, n), lambda i: (i, 0)),
        out_shape=jax.ShapeDtypeStruct((m, n), x.dtype),
    )(x, t, d, e)


# speedup vs baseline: 5.0839x; 1.0020x over previous
import functools

import jax
import jax.numpy as jnp
from jax.experimental import pallas as pl
from jax.experimental.pallas import tpu as pltpu

_CHUNK = 128


def _cumsum_body(x_ref, t_ref, d_ref, e_ref, o_ref):
    x = x_ref[...]
    t = t_ref[...]
    d = d_ref[...]
    e = e_ref[...]
    n = x.shape[1]
    nchunks = n // _CHUNK
    dot = functools.partial(
        jax.lax.dot, preferred_element_type=jnp.float32)
    ex = dot(dot(x, d), e)
    parts = [
        dot(x[:, i * _CHUNK:(i + 1) * _CHUNK], t) for i in range(nchunks)
    ]
    y = jnp.concatenate(parts, axis=1)
    o_ref[...] = y + ex


def kernel(x):
    m, n = x.shape
    r = 256
    nchunks = n // _CHUNK

    kk = jnp.arange(_CHUNK)
    t = (kk[:, None] <= kk[None, :]).astype(jnp.float32)
    krange = jnp.arange(n)
    crange = jnp.arange(nchunks)
    d = ((krange[:, None] // _CHUNK) < crange[None, :]).astype(jnp.float32)
    e = (crange[:, None] == (krange[None, :] // _CHUNK)).astype(jnp.float32)

    return pl.pallas_call(
        _cumsum_body,
        grid=(m // r,),
        in_specs=[
            pl.BlockSpec((r, n), lambda i: (i, 0)),
            pl.BlockSpec((_CHUNK, _CHUNK), lambda i: (0, 0)),
            pl.BlockSpec((n, nchunks), lambda i: (0, 0)),
            pl.BlockSpec((nchunks, n), lambda i: (0, 0)),
        ],
        out_specs=pl.BlockSpec((r, n), lambda i: (i, 0)),
        out_shape=jax.ShapeDtypeStruct((m, n), x.dtype),
        compiler_params=pltpu.CompilerParams(
            dimension_semantics=("parallel",)),
    )(x, t, d, e)


# speedup vs baseline: 5.3891x; 1.0600x over previous
import functools

import jax
import jax.numpy as jnp
from jax.experimental import pallas as pl
from jax.experimental.pallas import tpu as pltpu

_CHUNK = 128


def _cumsum_body(x_ref, t_ref, d_ref, e_ref, o_ref):
    x = x_ref[...]
    t = t_ref[...]
    d = d_ref[...]
    e = e_ref[...]
    n = x.shape[1]
    nchunks = n // _CHUNK
    dot = functools.partial(
        jax.lax.dot, preferred_element_type=jnp.float32)
    ex = dot(dot(x, d), e)
    parts = [
        dot(x[:, i * _CHUNK:(i + 1) * _CHUNK], t) for i in range(nchunks)
    ]
    y = jnp.concatenate(parts, axis=1)
    o_ref[...] = y + ex


def kernel(x):
    m, n = x.shape
    r = 128
    nchunks = n // _CHUNK

    kk = jnp.arange(_CHUNK)
    t = (kk[:, None] <= kk[None, :]).astype(jnp.float32)
    krange = jnp.arange(n)
    crange = jnp.arange(nchunks)
    d = ((krange[:, None] // _CHUNK) < crange[None, :]).astype(jnp.float32)
    e = (crange[:, None] == (krange[None, :] // _CHUNK)).astype(jnp.float32)

    return pl.pallas_call(
        _cumsum_body,
        grid=(m // r,),
        in_specs=[
            pl.BlockSpec((r, n), lambda i: (i, 0)),
            pl.BlockSpec((_CHUNK, _CHUNK), lambda i: (0, 0)),
            pl.BlockSpec((n, nchunks), lambda i: (0, 0)),
            pl.BlockSpec((nchunks, n), lambda i: (0, 0)),
        ],
        out_specs=pl.BlockSpec((r, n), lambda i: (i, 0)),
        out_shape=jax.ShapeDtypeStruct((m, n), x.dtype),
        compiler_params=pltpu.CompilerParams(
            dimension_semantics=("parallel",)),
    )(x, t, d, e)
